# Initial kernel scaffold; baseline (speedup 1.0000x reference)
import jax
import jax.numpy as jnp
from jax.experimental import pallas as pl


def kernel(features, edge_index, W1, b1, W2, b2, W3, b3):
    raise NotImplementedError("write your pallas kernel here")


# speedup vs baseline: 2.7596x; 2.7596x over previous
import functools

import jax
import jax.numpy as jnp
from jax import lax
from jax.experimental import pallas as pl
from jax.experimental.pallas import tpu as pltpu
from jax.experimental.pallas import tpu_sc as plsc

_N = 10000
_E = 320000
_NP = 10240
_EP = 327680
_CH = 128
_NT = 32
_NCH = (_EP // _NT) // _CH
_RPT = _NP // 16
_ZR = 64

_mesh = plsc.VectorSubcoreMesh(core_axis_name="c", subcore_axis_name="s")


def _degree_call(idx_rows):
    ipt_rows = (2 * _EP) // _CH // _NT
    zn = 2 * _NP // 16

    @functools.partial(
        pl.kernel,
        out_type=jax.ShapeDtypeStruct((2, 2 * _NP), jnp.float32),
        mesh=_mesh,
        scratch_types=[
            pltpu.VMEM_SHARED((2 * _NP,), jnp.float32),
            pltpu.VMEM((ipt_rows, _CH), jnp.int32),
            pltpu.VMEM((_CH,), jnp.float32),
            pltpu.VMEM((zn,), jnp.float32),
        ],
    )
    def k(idx_hbm, out_hbm, acc, idxv, ones_v, zbuf):
        cid = lax.axis_index("c")
        sid = lax.axis_index("s")
        wid = cid * 16 + sid

        @pl.loop(0, _CH, step=16)
        def _(i):
            ones_v[pl.ds(i, 16)] = jnp.ones((16,), jnp.float32)

        @pl.loop(0, zn, step=16)
        def _(i):
            zbuf[pl.ds(i, 16)] = jnp.zeros((16,), jnp.float32)

        pltpu.sync_copy(zbuf, acc.at[pl.ds(sid * zn, zn)])
        plsc.subcore_barrier()

        pltpu.sync_copy(idx_hbm.at[pl.ds(wid * ipt_rows, ipt_rows)], idxv)

        @pl.loop(0, ipt_rows)
        def _(j):
            pltpu.sync_copy(ones_v, acc.at[idxv.at[j]], add=True)

        plsc.subcore_barrier()
        pltpu.sync_copy(acc.at[pl.ds(sid * zn, zn)],
                        out_hbm.at[cid, pl.ds(sid * zn, zn)])

    return k(idx_rows)


def _aggregate_call(h, src_rows, dst_rows, d):
    @functools.partial(
        pl.kernel,
        out_type=jax.ShapeDtypeStruct((2, _NP, d), jnp.float32),
        mesh=_mesh,
        scratch_types=[
            pltpu.VMEM_SHARED((_NP, d), jnp.float32),
            pltpu.VMEM((_NCH, _CH), jnp.int32),
            pltpu.VMEM((_NCH, _CH), jnp.int32),
            pltpu.VMEM((_CH, d), jnp.float32),
            pltpu.VMEM((_ZR, d), jnp.float32),
        ],
    )
    def k(h_hbm, src_hbm, dst_hbm, out_hbm, acc, sidx, didx, rows, zbuf):
        cid = lax.axis_index("c")
        sid = lax.axis_index("s")
        wid = cid * 16 + sid

        @pl.loop(0, _ZR)
        def _(r):
            @pl.loop(0, d, step=16)
            def _(l):
                zbuf[r, pl.ds(l, 16)] = jnp.zeros((16,), jnp.float32)

        @pl.loop(0, _RPT, step=_ZR)
        def _(r):
            pltpu.sync_copy(zbuf, acc.at[pl.ds(sid * _RPT + r, _ZR)])

        plsc.subcore_barrier()

        pltpu.sync_copy(src_hbm.at[pl.ds(wid * _NCH, _NCH)], sidx)
        pltpu.sync_copy(dst_hbm.at[pl.ds(wid * _NCH, _NCH)], didx)

        @pl.loop(0, _NCH)
        def _(j):
            pltpu.sync_copy(h_hbm.at[sidx.at[j]], rows)
            pltpu.sync_copy(rows, acc.at[didx.at[j]], add=True)

        plsc.subcore_barrier()
        pltpu.sync_copy(acc.at[pl.ds(sid * _RPT, _RPT)],
                        out_hbm.at[cid, pl.ds(sid * _RPT, _RPT)])

    return k(h, src_rows, dst_rows)


_BN = 1024


def _tc_prep(featp, dout, din):
    def body(f_ref, do_ref, di_ref, h_ref, on_ref, in_ref):
        od = do_ref[:, 0:1] + do_ref[:, 1:2]
        idg = di_ref[:, 0:1] + di_ref[:, 1:2]
        on = lax.rsqrt(jnp.where(od > 0.0, od, 1.0))
        inn = lax.rsqrt(jnp.where(idg > 0.0, idg, 1.0))
        h_ref[...] = f_ref[...] * on
        on_ref[...] = on
        in_ref[...] = inn

    return pl.pallas_call(
        body,
        grid=(_NP // _BN,),
        in_specs=[
            pl.BlockSpec((_BN, 128), lambda i: (i, 0)),
            pl.BlockSpec((_BN, 2), lambda i: (i, 0)),
            pl.BlockSpec((_BN, 2), lambda i: (i, 0)),
        ],
        out_specs=[
            pl.BlockSpec((_BN, 128), lambda i: (i, 0)),
            pl.BlockSpec((_BN, 1), lambda i: (i, 0)),
            pl.BlockSpec((_BN, 1), lambda i: (i, 0)),
        ],
        out_shape=[
            jax.ShapeDtypeStruct((_NP, 128), jnp.float32),
            jax.ShapeDtypeStruct((_NP, 1), jnp.float32),
            jax.ShapeDtypeStruct((_NP, 1), jnp.float32),
        ],
    )(featp, dout, din)


def _log_softmax(y):
    m = jnp.max(y, axis=1, keepdims=True)
    return y - m - jnp.log(jnp.sum(jnp.exp(y - m), axis=1, keepdims=True))


def _tc_layer(p, inorm, onorm, W, b, Wnext=None):
    d_in = W.shape[0]
    d_mid = W.shape[1]
    d_out = 128 if Wnext is not None else d_mid

    def body(p_ref, in_ref, on_ref, W_ref, b_ref, *rest):
        if Wnext is not None:
            Wn_ref, o_ref = rest
        else:
            (o_ref,) = rest
        x = (p_ref[0] + p_ref[1]) * in_ref[...]
        y = jnp.dot(x, W_ref[...], preferred_element_type=jnp.float32)
        y = jnp.maximum(y + b_ref[...], 0.0)
        z = _log_softmax(y) * on_ref[...]
        if Wnext is not None:
            z = jnp.dot(z, Wn_ref[...], preferred_element_type=jnp.float32)
            z = jnp.concatenate(
                [z, jnp.zeros((z.shape[0], 128 - z.shape[1]), jnp.float32)],
                axis=1)
        o_ref[...] = z

    in_specs = [
        pl.BlockSpec((2, _BN, d_in), lambda i: (0, i, 0)),
        pl.BlockSpec((_BN, 1), lambda i: (i, 0)),
        pl.BlockSpec((_BN, 1), lambda i: (i, 0)),
        pl.BlockSpec((d_in, d_mid), lambda i: (0, 0)),
        pl.BlockSpec((1, d_mid), lambda i: (0, 0)),
    ]
    args = [p, inorm, onorm, W, b.reshape(1, -1)]
    if Wnext is not None:
        in_specs.append(
            pl.BlockSpec((d_mid, Wnext.shape[1]), lambda i: (0, 0)))
        args.append(Wnext)

    return pl.pallas_call(
        body,
        grid=(_NP // _BN,),
        in_specs=in_specs,
        out_specs=pl.BlockSpec((_BN, d_out), lambda i: (i, 0)),
        out_shape=jax.ShapeDtypeStruct((_NP, d_out), jnp.float32),
    )(*args)


def _tc_final(q, inorm, b3):
    d = b3.shape[0]
    dq = q.shape[-1]

    def body(q_ref, in_ref, b_ref, o_ref):
        x = (q_ref[0] + q_ref[1])[:, :d] * in_ref[...] + b_ref[...]
        o_ref[...] = _log_softmax(x)

    return pl.pallas_call(
        body,
        grid=(_NP // _BN,),
        in_specs=[
            pl.BlockSpec((2, _BN, dq), lambda i: (0, i, 0)),
            pl.BlockSpec((_BN, 1), lambda i: (i, 0)),
            pl.BlockSpec((1, d), lambda i: (0, 0)),
        ],
        out_specs=pl.BlockSpec((_BN, d), lambda i: (i, 0)),
        out_shape=jax.ShapeDtypeStruct((_NP, d), jnp.float32),
    )(q, inorm, b3.reshape(1, -1))


def kernel(features, edge_index, W1, b1, W2, b2, W3, b3):
    src = edge_index[0]
    dst = edge_index[1]
    pad = jnp.full((_EP - _E,), _N, jnp.int32)
    srcp = jnp.concatenate([src, pad]).reshape(_EP // _CH, _CH)
    dstp = jnp.concatenate([dst, pad]).reshape(_EP // _CH, _CH)
    degidx = jnp.concatenate([srcp, dstp + _NP], axis=0)

    featp = jnp.concatenate(
        [features, jnp.zeros((_NP - _N, features.shape[1]), jnp.float32)], axis=0)

    degp = _degree_call(degidx)
    dout = degp[:, :_NP].T
    din = degp[:, _NP:].T

    h1s, onorm, inorm = _tc_prep(featp, dout, din)
    p1 = _aggregate_call(h1s, srcp, dstp, 128)
    h2s = _tc_layer(p1, inorm, onorm, W1, b1)
    p2 = _aggregate_call(h2s, srcp, dstp, 128)
    t3 = _tc_layer(p2, inorm, onorm, W2, b2, Wnext=W3)
    p3 = _aggregate_call(t3, srcp, dstp, 128)
    out = _tc_final(p3, inorm, b3)
    return out[:_N]


# speedup vs baseline: 3.3881x; 1.2277x over previous
import functools

import jax
import jax.numpy as jnp
from jax import lax
from jax.experimental import pallas as pl
from jax.experimental.pallas import tpu as pltpu
from jax.experimental.pallas import tpu_sc as plsc

_N = 10000
_E = 320000
_NP = 10240
_EP = 327680
_CH = 80
_NT = 32
_NCH = (_EP // _NT) // _CH
_RPT = _NP // 16

_mesh = plsc.VectorSubcoreMesh(core_axis_name="c", subcore_axis_name="s")


def _degree_call(idx_rows):
    ipt_rows = (2 * _EP) // _CH // _NT
    zn = 2 * _NP // 16

    @functools.partial(
        pl.kernel,
        out_type=jax.ShapeDtypeStruct((2, 2 * _NP), jnp.float32),
        mesh=_mesh,
        scratch_types=[
            pltpu.VMEM_SHARED((2 * _NP,), jnp.float32),
            pltpu.VMEM((ipt_rows, _CH), jnp.int32),
            pltpu.VMEM((_CH,), jnp.float32),
            pltpu.VMEM((zn,), jnp.float32),
        ],
    )
    def k(idx_hbm, out_hbm, acc, idxv, ones_v, zbuf):
        cid = lax.axis_index("c")
        sid = lax.axis_index("s")
        wid = cid * 16 + sid

        @pl.loop(0, _CH, step=16)
        def _(i):
            ones_v[pl.ds(i, 16)] = jnp.ones((16,), jnp.float32)

        @pl.loop(0, zn, step=16)
        def _(i):
            zbuf[pl.ds(i, 16)] = jnp.zeros((16,), jnp.float32)

        pltpu.sync_copy(zbuf, acc.at[pl.ds(sid * zn, zn)])
        plsc.subcore_barrier()

        pltpu.sync_copy(idx_hbm.at[pl.ds(wid * ipt_rows, ipt_rows)], idxv)

        @pl.loop(0, ipt_rows)
        def _(j):
            pltpu.sync_copy(ones_v, acc.at[idxv.at[j]], add=True)

        plsc.subcore_barrier()
        pltpu.sync_copy(acc.at[pl.ds(sid * zn, zn)],
                        out_hbm.at[cid, pl.ds(sid * zn, zn)])

    return k(idx_rows)


_NB = 4
_NGRP = _NCH // _NB


def _aggregate_call(h, src_rows, dst_rows, d):
    @functools.partial(
        pl.kernel,
        out_type=jax.ShapeDtypeStruct((2, _NP, d), jnp.float32),
        mesh=_mesh,
        scratch_types=[
            pltpu.VMEM_SHARED((_NP, d), jnp.float32),
            pltpu.VMEM((2, _NB, _CH), jnp.int32),
            pltpu.VMEM((2, _NB, _CH), jnp.int32),
            pltpu.VMEM((_NB, _CH, d), jnp.float32),
        ] + [pltpu.SemaphoreType.DMA] * (2 * _NB + 1),
    )
    def k(h_hbm, src_hbm, dst_hbm, out_hbm, acc, sidx, didx, rows, *sems):
        sem_g = sems[:_NB]
        sem_s = sems[_NB:2 * _NB]
        sem_i = sems[2 * _NB]
        cid = lax.axis_index("c")
        sid = lax.axis_index("s")
        wid = cid * 16 + sid
        tb = wid * _NCH

        @pl.loop(0, _CH)
        def _(r):
            @pl.loop(0, d, step=16)
            def _(l):
                rows[0, r, pl.ds(l, 16)] = jnp.zeros((16,), jnp.float32)

        @pl.loop(0, _RPT, step=_CH)
        def _(r):
            pltpu.sync_copy(rows.at[0], acc.at[pl.ds(sid * _RPT + r, _CH)])

        plsc.subcore_barrier()

        def idx_load(grp, par):
            pltpu.async_copy(src_rows_slice(grp), sidx.at[par], sem_i)
            pltpu.async_copy(dst_rows_slice(grp), didx.at[par], sem_i)

        def idx_wait(grp, par):
            pltpu.make_async_copy(src_rows_slice(grp), sidx.at[par],
                                  sem_i).wait()
            pltpu.make_async_copy(dst_rows_slice(grp), didx.at[par],
                                  sem_i).wait()

        def src_rows_slice(grp):
            return src_hbm.at[pl.ds(tb + grp * _NB, _NB)]

        def dst_rows_slice(grp):
            return dst_hbm.at[pl.ds(tb + grp * _NB, _NB)]

        def gather_start(par, b):
            pltpu.async_copy(h_hbm.at[sidx.at[par, b]], rows.at[b],
                             sem_g[b])

        def gather_wait(par, b):
            pltpu.make_async_copy(h_hbm.at[sidx.at[par, b]], rows.at[b],
                                  sem_g[b]).wait()

        def scatter_start(par, b):
            pltpu.async_copy(rows.at[b], acc.at[didx.at[par, b]],
                             sem_s[b], add=True)

        def scatter_wait(par, b):
            pltpu.make_async_copy(rows.at[b], acc.at[didx.at[par, b]],
                                  sem_s[b]).wait()

        pltpu.sync_copy(src_rows_slice(0), sidx.at[0])
        pltpu.sync_copy(dst_rows_slice(0), didx.at[0])
        for b in range(_NB):
            gather_start(0, b)
        idx_load(1, 1)

        @pl.loop(0, _NGRP)
        def _(gi):
            par = lax.rem(gi, 2)
            nxt_par = lax.rem(gi + 1, 2)
            for b in range(_NB):
                gather_wait(par, b)
                scatter_start(par, b)

            @pl.when(gi + 1 < _NGRP)
            def _():
                idx_wait(gi + 1, nxt_par)

            for b in range(_NB):
                scatter_wait(par, b)

                @pl.when(gi + 1 < _NGRP)
                def _():
                    gather_start(nxt_par, b)

            @pl.when(gi + 2 < _NGRP)
            def _():
                idx_load(gi + 2, par)

        plsc.subcore_barrier()
        pltpu.sync_copy(acc.at[pl.ds(sid * _RPT, _RPT)],
                        out_hbm.at[cid, pl.ds(sid * _RPT, _RPT)])

    return k(h, src_rows, dst_rows)


_BN = 1024


def _tc_prep(featp, dout, din):
    def body(f_ref, do_ref, di_ref, h_ref, on_ref, in_ref):
        od = do_ref[:, 0:1] + do_ref[:, 1:2]
        idg = di_ref[:, 0:1] + di_ref[:, 1:2]
        on = lax.rsqrt(jnp.where(od > 0.0, od, 1.0))
        inn = lax.rsqrt(jnp.where(idg > 0.0, idg, 1.0))
        h_ref[...] = f_ref[...] * on
        on_ref[...] = on
        in_ref[...] = inn

    return pl.pallas_call(
        body,
        grid=(_NP // _BN,),
        in_specs=[
            pl.BlockSpec((_BN, 128), lambda i: (i, 0)),
            pl.BlockSpec((_BN, 2), lambda i: (i, 0)),
            pl.BlockSpec((_BN, 2), lambda i: (i, 0)),
        ],
        out_specs=[
            pl.BlockSpec((_BN, 128), lambda i: (i, 0)),
            pl.BlockSpec((_BN, 1), lambda i: (i, 0)),
            pl.BlockSpec((_BN, 1), lambda i: (i, 0)),
        ],
        out_shape=[
            jax.ShapeDtypeStruct((_NP, 128), jnp.float32),
            jax.ShapeDtypeStruct((_NP, 1), jnp.float32),
            jax.ShapeDtypeStruct((_NP, 1), jnp.float32),
        ],
    )(featp, dout, din)


def _log_softmax(y):
    m = jnp.max(y, axis=1, keepdims=True)
    return y - m - jnp.log(jnp.sum(jnp.exp(y - m), axis=1, keepdims=True))


def _tc_layer(p, inorm, onorm, W, b, Wnext=None):
    d_in = W.shape[0]
    d_mid = W.shape[1]
    d_out = 128 if Wnext is not None else d_mid

    def body(p_ref, in_ref, on_ref, W_ref, b_ref, *rest):
        if Wnext is not None:
            Wn_ref, o_ref = rest
        else:
            (o_ref,) = rest
        x = (p_ref[0] + p_ref[1]) * in_ref[...]
        y = jnp.dot(x, W_ref[...], preferred_element_type=jnp.float32)
        y = jnp.maximum(y + b_ref[...], 0.0)
        z = _log_softmax(y) * on_ref[...]
        if Wnext is not None:
            z = jnp.dot(z, Wn_ref[...], preferred_element_type=jnp.float32)
            z = jnp.concatenate(
                [z, jnp.zeros((z.shape[0], 128 - z.shape[1]), jnp.float32)],
                axis=1)
        o_ref[...] = z

    in_specs = [
        pl.BlockSpec((2, _BN, d_in), lambda i: (0, i, 0)),
        pl.BlockSpec((_BN, 1), lambda i: (i, 0)),
        pl.BlockSpec((_BN, 1), lambda i: (i, 0)),
        pl.BlockSpec((d_in, d_mid), lambda i: (0, 0)),
        pl.BlockSpec((1, d_mid), lambda i: (0, 0)),
    ]
    args = [p, inorm, onorm, W, b.reshape(1, -1)]
    if Wnext is not None:
        in_specs.append(
            pl.BlockSpec((d_mid, Wnext.shape[1]), lambda i: (0, 0)))
        args.append(Wnext)

    return pl.pallas_call(
        body,
        grid=(_NP // _BN,),
        in_specs=in_specs,
        out_specs=pl.BlockSpec((_BN, d_out), lambda i: (i, 0)),
        out_shape=jax.ShapeDtypeStruct((_NP, d_out), jnp.float32),
    )(*args)


def _tc_final(q, inorm, b3):
    d = b3.shape[0]
    dq = q.shape[-1]

    def body(q_ref, in_ref, b_ref, o_ref):
        x = (q_ref[0] + q_ref[1])[:, :d] * in_ref[...] + b_ref[...]
        o_ref[...] = _log_softmax(x)

    return pl.pallas_call(
        body,
        grid=(_NP // _BN,),
        in_specs=[
            pl.BlockSpec((2, _BN, dq), lambda i: (0, i, 0)),
            pl.BlockSpec((_BN, 1), lambda i: (i, 0)),
            pl.BlockSpec((1, d), lambda i: (0, 0)),
        ],
        out_specs=pl.BlockSpec((_BN, d), lambda i: (i, 0)),
        out_shape=jax.ShapeDtypeStruct((_NP, d), jnp.float32),
    )(q, inorm, b3.reshape(1, -1))


def kernel(features, edge_index, W1, b1, W2, b2, W3, b3):
    src = edge_index[0]
    dst = edge_index[1]
    pad = jnp.full((_EP - _E,), _N, jnp.int32)
    srcp = jnp.concatenate([src, pad]).reshape(_EP // _CH, _CH)
    dstp = jnp.concatenate([dst, pad]).reshape(_EP // _CH, _CH)
    degidx = jnp.concatenate([srcp, dstp + _NP], axis=0)

    featp = jnp.concatenate(
        [features, jnp.zeros((_NP - _N, features.shape[1]), jnp.float32)], axis=0)

    degp = _degree_call(degidx)
    dout = degp[:, :_NP].T
    din = degp[:, _NP:].T

    h1s, onorm, inorm = _tc_prep(featp, dout, din)
    p1 = _aggregate_call(h1s, srcp, dstp, 128)
    h2s = _tc_layer(p1, inorm, onorm, W1, b1)
    p2 = _aggregate_call(h2s, srcp, dstp, 128)
    t3 = _tc_layer(p2, inorm, onorm, W2, b2, Wnext=W3)
    p3 = _aggregate_call(t3, srcp, dstp, 128)
    out = _tc_final(p3, inorm, b3)
    return out[:_N]


# speedup vs baseline: 3.6489x; 1.0770x over previous
import functools

import jax
import jax.numpy as jnp
from jax import lax
from jax.experimental import pallas as pl
from jax.experimental.pallas import tpu as pltpu
from jax.experimental.pallas import tpu_sc as plsc

_N = 10000
_E = 320000
_NP = 10240
_EP = 327680
_CH = 80
_NT = 32
_NCH = (_EP // _NT) // _CH
_RPT = _NP // 16

_mesh = plsc.VectorSubcoreMesh(core_axis_name="c", subcore_axis_name="s")


def _degree_call(idx_rows):
    ipt_rows = (2 * _EP) // _CH // _NT
    zn = 2 * _NP // 16

    @functools.partial(
        pl.kernel,
        out_type=jax.ShapeDtypeStruct((2, 2 * _NP), jnp.float32),
        mesh=_mesh,
        scratch_types=[
            pltpu.VMEM_SHARED((2 * _NP,), jnp.float32),
            pltpu.VMEM((ipt_rows, _CH), jnp.int32),
            pltpu.VMEM((_CH,), jnp.float32),
            pltpu.VMEM((zn,), jnp.float32),
        ],
    )
    def k(idx_hbm, out_hbm, acc, idxv, ones_v, zbuf):
        cid = lax.axis_index("c")
        sid = lax.axis_index("s")
        wid = cid * 16 + sid

        @pl.loop(0, _CH, step=16)
        def _(i):
            ones_v[pl.ds(i, 16)] = jnp.ones((16,), jnp.float32)

        @pl.loop(0, zn, step=16)
        def _(i):
            zbuf[pl.ds(i, 16)] = jnp.zeros((16,), jnp.float32)

        pltpu.sync_copy(zbuf, acc.at[pl.ds(sid * zn, zn)])
        plsc.subcore_barrier()

        pltpu.sync_copy(idx_hbm.at[pl.ds(wid * ipt_rows, ipt_rows)], idxv)

        @pl.loop(0, ipt_rows)
        def _(j):
            pltpu.sync_copy(ones_v, acc.at[idxv.at[j]], add=True)

        plsc.subcore_barrier()
        pltpu.sync_copy(acc.at[pl.ds(sid * zn, zn)],
                        out_hbm.at[cid, pl.ds(sid * zn, zn)])

    return k(idx_rows)


_NB = 4
_NCH0 = 208
_NCH1 = (_EP // _CH - 16 * _NCH0) // 16
_NGRP0 = _NCH0 // _NB
_NGRP1 = _NCH1 // _NB


def _aggregate_call(h, src_rows, dst_rows, d):
    @functools.partial(
        pl.kernel,
        out_type=jax.ShapeDtypeStruct((2, _NP, d), jnp.float32),
        mesh=_mesh,
        scratch_types=[
            pltpu.VMEM_SHARED((_NP, d), jnp.float32),
            pltpu.VMEM((2, _NB, _CH), jnp.int32),
            pltpu.VMEM((2, _NB, _CH), jnp.int32),
            pltpu.VMEM((_NB, _CH, d), jnp.float32),
        ] + [pltpu.SemaphoreType.DMA] * (2 * _NB + 1),
    )
    def k(h_hbm, src_hbm, dst_hbm, out_hbm, acc, sidx, didx, rows, *sems):
        sem_g = sems[:_NB]
        sem_s = sems[_NB:2 * _NB]
        sem_i = sems[2 * _NB]
        cid = lax.axis_index("c")
        sid = lax.axis_index("s")
        tb = jnp.where(cid == 0, sid * _NCH0, 16 * _NCH0 + sid * _NCH1)
        ngrp = jnp.where(cid == 0, _NGRP0, _NGRP1)

        @pl.loop(0, _CH)
        def _(r):
            @pl.loop(0, d, step=16)
            def _(l):
                rows[0, r, pl.ds(l, 16)] = jnp.zeros((16,), jnp.float32)

        @pl.loop(0, _RPT, step=_CH)
        def _(r):
            pltpu.sync_copy(rows.at[0], acc.at[pl.ds(sid * _RPT + r, _CH)])

        plsc.subcore_barrier()

        def idx_load(grp, par):
            pltpu.async_copy(src_rows_slice(grp), sidx.at[par], sem_i)
            pltpu.async_copy(dst_rows_slice(grp), didx.at[par], sem_i)

        def idx_wait(grp, par):
            pltpu.make_async_copy(src_rows_slice(grp), sidx.at[par],
                                  sem_i).wait()
            pltpu.make_async_copy(dst_rows_slice(grp), didx.at[par],
                                  sem_i).wait()

        def src_rows_slice(grp):
            return src_hbm.at[pl.ds(tb + grp * _NB, _NB)]

        def dst_rows_slice(grp):
            return dst_hbm.at[pl.ds(tb + grp * _NB, _NB)]

        def gather_start(par, b):
            pltpu.async_copy(h_hbm.at[sidx.at[par, b]], rows.at[b],
                             sem_g[b])

        def gather_wait(par, b):
            pltpu.make_async_copy(h_hbm.at[sidx.at[par, b]], rows.at[b],
                                  sem_g[b]).wait()

        def scatter_start(par, b):
            pltpu.async_copy(rows.at[b], acc.at[didx.at[par, b]],
                             sem_s[b], add=True)

        def scatter_wait(par, b):
            pltpu.make_async_copy(rows.at[b], acc.at[didx.at[par, b]],
                                  sem_s[b]).wait()

        pltpu.sync_copy(src_rows_slice(0), sidx.at[0])
        pltpu.sync_copy(dst_rows_slice(0), didx.at[0])
        for b in range(_NB):
            gather_start(0, b)
        idx_load(1, 1)

        @pl.loop(0, _NGRP0)
        def _(gi):
            @pl.when(gi < ngrp)
            def _():
                par = lax.rem(gi, 2)
                nxt_par = lax.rem(gi + 1, 2)
                for b in range(_NB):
                    gather_wait(par, b)
                    scatter_start(par, b)

                @pl.when(gi + 1 < ngrp)
                def _():
                    idx_wait(gi + 1, nxt_par)

                for b in range(_NB):
                    scatter_wait(par, b)

                    @pl.when(gi + 1 < ngrp)
                    def _():
                        gather_start(nxt_par, b)

                @pl.when(gi + 2 < ngrp)
                def _():
                    idx_load(gi + 2, par)

        plsc.subcore_barrier()
        pltpu.sync_copy(acc.at[pl.ds(sid * _RPT, _RPT)],
                        out_hbm.at[cid, pl.ds(sid * _RPT, _RPT)])

    return k(h, src_rows, dst_rows)


_BN = 1024


def _tc_prep(featp, dout, din):
    def body(f_ref, do_ref, di_ref, h_ref, on_ref, in_ref):
        od = do_ref[:, 0:1] + do_ref[:, 1:2]
        idg = di_ref[:, 0:1] + di_ref[:, 1:2]
        on = lax.rsqrt(jnp.where(od > 0.0, od, 1.0))
        inn = lax.rsqrt(jnp.where(idg > 0.0, idg, 1.0))
        h_ref[...] = f_ref[...] * on
        on_ref[...] = on
        in_ref[...] = inn

    return pl.pallas_call(
        body,
        grid=(_NP // _BN,),
        in_specs=[
            pl.BlockSpec((_BN, 128), lambda i: (i, 0)),
            pl.BlockSpec((_BN, 2), lambda i: (i, 0)),
            pl.BlockSpec((_BN, 2), lambda i: (i, 0)),
        ],
        out_specs=[
            pl.BlockSpec((_BN, 128), lambda i: (i, 0)),
            pl.BlockSpec((_BN, 1), lambda i: (i, 0)),
            pl.BlockSpec((_BN, 1), lambda i: (i, 0)),
        ],
        out_shape=[
            jax.ShapeDtypeStruct((_NP, 128), jnp.float32),
            jax.ShapeDtypeStruct((_NP, 1), jnp.float32),
            jax.ShapeDtypeStruct((_NP, 1), jnp.float32),
        ],
    )(featp, dout, din)


def _log_softmax(y):
    m = jnp.max(y, axis=1, keepdims=True)
    return y - m - jnp.log(jnp.sum(jnp.exp(y - m), axis=1, keepdims=True))


def _tc_layer(p, inorm, onorm, W, b, Wnext=None):
    d_in = W.shape[0]
    d_mid = W.shape[1]
    d_out = 128 if Wnext is not None else d_mid

    def body(p_ref, in_ref, on_ref, W_ref, b_ref, *rest):
        if Wnext is not None:
            Wn_ref, o_ref = rest
        else:
            (o_ref,) = rest
        x = (p_ref[0] + p_ref[1]) * in_ref[...]
        y = jnp.dot(x, W_ref[...], preferred_element_type=jnp.float32)
        y = jnp.maximum(y + b_ref[...], 0.0)
        z = _log_softmax(y) * on_ref[...]
        if Wnext is not None:
            z = jnp.dot(z, Wn_ref[...], preferred_element_type=jnp.float32)
            z = jnp.concatenate(
                [z, jnp.zeros((z.shape[0], 128 - z.shape[1]), jnp.float32)],
                axis=1)
        o_ref[...] = z

    in_specs = [
        pl.BlockSpec((2, _BN, d_in), lambda i: (0, i, 0)),
        pl.BlockSpec((_BN, 1), lambda i: (i, 0)),
        pl.BlockSpec((_BN, 1), lambda i: (i, 0)),
        pl.BlockSpec((d_in, d_mid), lambda i: (0, 0)),
        pl.BlockSpec((1, d_mid), lambda i: (0, 0)),
    ]
    args = [p, inorm, onorm, W, b.reshape(1, -1)]
    if Wnext is not None:
        in_specs.append(
            pl.BlockSpec((d_mid, Wnext.shape[1]), lambda i: (0, 0)))
        args.append(Wnext)

    return pl.pallas_call(
        body,
        grid=(_NP // _BN,),
        in_specs=in_specs,
        out_specs=pl.BlockSpec((_BN, d_out), lambda i: (i, 0)),
        out_shape=jax.ShapeDtypeStruct((_NP, d_out), jnp.float32),
    )(*args)


def _tc_final(q, inorm, b3):
    d = b3.shape[0]
    dq = q.shape[-1]

    def body(q_ref, in_ref, b_ref, o_ref):
        x = (q_ref[0] + q_ref[1])[:, :d] * in_ref[...] + b_ref[...]
        o_ref[...] = _log_softmax(x)

    return pl.pallas_call(
        body,
        grid=(_NP // _BN,),
        in_specs=[
            pl.BlockSpec((2, _BN, dq), lambda i: (0, i, 0)),
            pl.BlockSpec((_BN, 1), lambda i: (i, 0)),
            pl.BlockSpec((1, d), lambda i: (0, 0)),
        ],
        out_specs=pl.BlockSpec((_BN, d), lambda i: (i, 0)),
        out_shape=jax.ShapeDtypeStruct((_NP, d), jnp.float32),
    )(q, inorm, b3.reshape(1, -1))


def kernel(features, edge_index, W1, b1, W2, b2, W3, b3):
    src = edge_index[0]
    dst = edge_index[1]
    pad = jnp.full((_EP - _E,), _N, jnp.int32)
    srcp = jnp.concatenate([src, pad]).reshape(_EP // _CH, _CH)
    dstp = jnp.concatenate([dst, pad]).reshape(_EP // _CH, _CH)
    degidx = jnp.concatenate([srcp, dstp + _NP], axis=0)

    featp = jnp.concatenate(
        [features, jnp.zeros((_NP - _N, features.shape[1]), jnp.float32)], axis=0)

    degp = _degree_call(degidx)
    dout = degp[:, :_NP].T
    din = degp[:, _NP:].T

    h1s, onorm, inorm = _tc_prep(featp, dout, din)
    p1 = _aggregate_call(h1s, srcp, dstp, 128)
    h2s = _tc_layer(p1, inorm, onorm, W1, b1)
    p2 = _aggregate_call(h2s, srcp, dstp, 128)
    t3 = _tc_layer(p2, inorm, onorm, W2, b2, Wnext=W3)
    p3 = _aggregate_call(t3, srcp, dstp, 128)
    out = _tc_final(p3, inorm, b3)
    return out[:_N]
